# baseline (speedup 1.0000x reference)
import functools

import jax
import jax.numpy as jnp
from jax import lax
from jax.experimental import pallas as pl
from jax.experimental.pallas import tpu as pltpu
from jax.experimental.pallas import tpu_sc as plsc

N = 16384
NB = 8
NPB = N // NB
KNN = 16
C = 64


_RT = 128


def _knn_body(xi_ref, yi_ref, zi_ref, xj_ref, yj_ref, zj_ref, idx_ref, neg_ref):
    b = pl.program_id(0)
    t = pl.program_id(1)
    xi = xi_ref[...]
    yi = yi_ref[...]
    zi = zi_ref[...]
    xj = xj_ref[...]
    yj = yj_ref[...]
    zj = zj_ref[...]
    dx = xi - xj
    dy = yi - yj
    dz = zi - zj
    d = (dx * dx + dy * dy) + dz * dz
    row = lax.broadcasted_iota(jnp.int32, (_RT, NPB), 0) + t * _RT
    col = lax.broadcasted_iota(jnp.int32, (_RT, NPB), 1)
    d = d + jnp.where(row == col, jnp.float32(1e10), jnp.float32(0.0))
    neg_ref[...] = -d

    lane = lax.broadcasted_iota(jnp.int32, (_RT, KNN), 1)
    colf = col.astype(jnp.float32)

    def it(i, sel):
        neg = neg_ref[...]
        m = jnp.max(neg, axis=1, keepdims=True)
        cand = jnp.where(neg == m, colf, jnp.float32(NPB))
        a = jnp.min(cand, axis=1, keepdims=True)
        neg_ref[...] = jnp.where(colf == a, jnp.float32(-jnp.inf), neg)
        return jnp.where(lane == i, a, sel)

    self0 = jnp.zeros((_RT, KNN), jnp.float32)
    sel = lax.fori_loop(0, KNN, it, self0)
    idx_ref[...] = sel.astype(jnp.int32) + b * NPB


def _knn(xi, yi, zi, xj, yj, zj, nb):
    bi = pl.BlockSpec((None, _RT, 1), lambda b, t: (b, t, 0))
    bj = pl.BlockSpec((None, 1, NPB), lambda b, t: (b, 0, 0))
    bo = pl.BlockSpec((None, _RT, KNN), lambda b, t: (b, t, 0))
    return pl.pallas_call(
        _knn_body,
        grid=(nb, NPB // _RT),
        in_specs=[bi, bi, bi, bj, bj, bj],
        out_specs=bo,
        out_shape=jax.ShapeDtypeStruct((nb, NPB, KNN), jnp.int32),
        scratch_shapes=[pltpu.VMEM((_RT, NPB), jnp.float32)],
    )(xi, yi, zi, xj, yj, zj)


_RM = 1024


def _uv_body(f_ref, w_ref, b_ref, u_ref, v_ref):
    f = f_ref[...]
    w = w_ref[...]
    wa = w[:, :C]
    wb = w[:, C:]
    dn = (((1,), (1,)), ((), ()))
    u_ref[...] = lax.dot_general(f, wa - wb, dn,
                                 preferred_element_type=jnp.float32) + b_ref[...]
    v_ref[...] = lax.dot_general(f, wb, dn, preferred_element_type=jnp.float32)


def _uv(f, w, bvec):
    nr = f.shape[0]
    return pl.pallas_call(
        _uv_body,
        grid=(nr // _RM,),
        in_specs=[pl.BlockSpec((_RM, C), lambda i: (i, 0)),
                  pl.BlockSpec((C, 2 * C), lambda i: (0, 0)),
                  pl.BlockSpec((1, C), lambda i: (0, 0))],
        out_specs=[pl.BlockSpec((_RM, C), lambda i: (i, 0)),
                   pl.BlockSpec((_RM, C), lambda i: (i, 0))],
        out_shape=[jax.ShapeDtypeStruct((nr, C), jnp.float32),
                   jax.ShapeDtypeStruct((nr, C), jnp.float32)],
    )(f, w, bvec.reshape(1, C))


_NC = 2
_NS = 16
_NW = _NC * _NS
_CHR = 32
_IW = 64
_NG = _CHR * KNN // _IW
_NBUF = 2


def _make_sc(nrows, add_base):
    _ROWS_W = nrows // _NW
    _NCHUNK = _ROWS_W // _CHR
    assert _NCHUNK % _NBUF == 0
    mesh = plsc.VectorSubcoreMesh(core_axis_name="c", subcore_axis_name="s",
                                  num_cores=_NC)
    scratch = [
        pltpu.VMEM((_NBUF, _NG, _IW), jnp.int32),
        pltpu.VMEM((_NBUF, _CHR * KNN, C), jnp.float32),
        pltpu.VMEM((_NBUF, _CHR, C), jnp.float32),
        pltpu.VMEM((_NBUF, _CHR, C), jnp.float32),
        pltpu.VMEM((_NBUF, _CHR, C), jnp.float32),
    ]
    if add_base:
        scratch.append(pltpu.VMEM((_NBUF, _CHR, C), jnp.float32))
    scratch += [pltpu.SemaphoreType.DMA] * (3 * _NBUF)

    def body(u_hbm, v_hbm, f_hbm, idx_hbm, *rest):
        if add_base:
            (base_hbm, out_hbm, idx_v, rows_v, u_v, f_v, o_v, base_v,
             *sems) = rest
        else:
            (out_hbm, idx_v, rows_v, u_v, f_v, o_v, *sems) = rest
        gsem = sems[:_NBUF]
        lsem = sems[_NBUF:2 * _NBUF]
        osem = sems[2 * _NBUF:]
        wid = lax.axis_index("s") * _NC + lax.axis_index("c")
        r0w = wid * _ROWS_W

        def _in_copies(cidx, b):
            r0 = pl.multiple_of(r0w + cidx * _CHR, _CHR)
            cps = [pltpu.make_async_copy(v_hbm.at[idx_v.at[b, g]],
                                         rows_v.at[b, pl.ds(g * _IW, _IW)],
                                         gsem[b])
                   for g in range(_NG)]
            cps.append(pltpu.make_async_copy(u_hbm.at[pl.ds(r0, _CHR)],
                                             u_v.at[b], lsem[b]))
            cps.append(pltpu.make_async_copy(f_hbm.at[pl.ds(r0, _CHR)],
                                             f_v.at[b], lsem[b]))
            if add_base:
                cps.append(pltpu.make_async_copy(
                    base_hbm.at[pl.ds(r0, _CHR)], base_v.at[b], lsem[b]))
            return cps

        def stage(cidx, b):
            r0 = pl.multiple_of(r0w + cidx * _CHR, _CHR)
            pltpu.sync_copy(
                idx_hbm.at[pl.ds(pl.multiple_of(r0 * KNN // _IW, 8), _NG)],
                idx_v.at[b])
            for cp in _in_copies(cidx, b):
                cp.start()

        def compute(cidx, b):
            for cp in _in_copies(cidx, b):
                cp.wait()

            def rowfn(r2, rc):
                for u in range(2):
                    r = r2 * 2 + u
                    rk = r * KNN
                    for j in range(C // 16):
                        sl = pl.ds(j * 16, 16)
                        m = rows_v[b, rk, sl]
                        for k in range(1, KNN):
                            m = jnp.maximum(m, rows_v[b, rk + k, sl])
                        agg = jnp.maximum(u_v[b, r, sl] + m, 0.0)
                        fn = jnp.maximum(f_v[b, r, sl] + agg, 0.0)
                        if add_base:
                            fn = fn + base_v[b, r, sl]
                        o_v[b, r, sl] = fn
                return rc

            lax.fori_loop(0, _CHR // 2, rowfn, 0)

        def _out_copy(cidx, b):
            r0 = pl.multiple_of(r0w + cidx * _CHR, _CHR)
            return pltpu.make_async_copy(o_v.at[b],
                                         out_hbm.at[pl.ds(r0, _CHR)], osem[b])

        for b in range(_NBUF):
            stage(b, b)

        def group(g0, carry):
            for b in range(_NBUF):
                cidx = g0 * _NBUF + b

                @pl.when(cidx >= _NBUF)
                def _():
                    _out_copy(cidx - _NBUF, b).wait()
                compute(cidx, b)
                _out_copy(cidx, b).start()

                @pl.when(cidx + _NBUF < _NCHUNK)
                def _():
                    stage(cidx + _NBUF, b)
            return carry

        lax.fori_loop(0, _NCHUNK // _NBUF, group, 0)
        for b in range(_NBUF):
            _out_copy(_NCHUNK - _NBUF + b, b).wait()

    return pl.kernel(
        body,
        mesh=mesh,
        out_type=jax.ShapeDtypeStruct((nrows, C), jnp.float32),
        scratch_types=scratch,
        compiler_params=pltpu.CompilerParams(use_tc_tiling_on_sc=False),
    )


_G = 1


def kernel(voxel_coords, pillar_features, W1, b1, W2, b2):
    nbg = NB // _G
    ng = N // _G
    _sc_l1 = _make_sc(ng, False)
    _sc_l2 = _make_sc(ng, True)
    pos = voxel_coords[:, 1:4]
    p3 = pos.reshape(NB, NPB, 3)
    xi = p3[:, :, 0:1]
    yi = p3[:, :, 1:2]
    zi = p3[:, :, 2:3]
    xj = xi.reshape(NB, 1, NPB)
    yj = yi.reshape(NB, 1, NPB)
    zj = zi.reshape(NB, 1, NPB)
    outs = []
    for g in range(_G):
        bs = slice(g * nbg, (g + 1) * nbg)
        idx = _knn(xi[bs], yi[bs], zi[bs], xj[bs], yj[bs], zj[bs], nbg)
        idx2d = idx.reshape(ng * KNN // _IW, _IW)
        f0 = pillar_features[g * ng:(g + 1) * ng]
        u1, v1 = _uv(f0, W1, b1)
        f1 = _sc_l1(u1, v1, f0, idx2d)
        u2, v2 = _uv(f1, W2, b2)
        outs.append(_sc_l2(u2, v2, f1, idx2d, f0))
    return jnp.concatenate(outs, axis=0)


# speedup vs baseline: 1.1792x; 1.1792x over previous
import functools

import jax
import jax.numpy as jnp
from jax import lax
from jax.experimental import pallas as pl
from jax.experimental.pallas import tpu as pltpu
from jax.experimental.pallas import tpu_sc as plsc

N = 16384
NB = 8
NPB = N // NB
KNN = 16
C = 64


_RT = 256


def _knn_body(xi_ref, yi_ref, zi_ref, xj_ref, yj_ref, zj_ref, idx_ref, neg_ref):
    b = pl.program_id(0)
    t = pl.program_id(1)
    xi = xi_ref[...]
    yi = yi_ref[...]
    zi = zi_ref[...]
    xj = xj_ref[...]
    yj = yj_ref[...]
    zj = zj_ref[...]
    dx = xi - xj
    dy = yi - yj
    dz = zi - zj
    d = (dx * dx + dy * dy) + dz * dz
    row = lax.broadcasted_iota(jnp.int32, (_RT, NPB), 0) + t * _RT
    col = lax.broadcasted_iota(jnp.int32, (_RT, NPB), 1)
    d = d + jnp.where(row == col, jnp.float32(1e10), jnp.float32(0.0))
    neg_ref[...] = -d

    lane = lax.broadcasted_iota(jnp.int32, (_RT, KNN), 1)
    colf = col.astype(jnp.float32)

    def it(i, sel):
        neg = neg_ref[...]
        m = jnp.max(neg, axis=1, keepdims=True)
        cand = jnp.where(neg == m, colf, jnp.float32(NPB))
        a = jnp.min(cand, axis=1, keepdims=True)
        neg_ref[...] = jnp.where(colf == a, jnp.float32(-jnp.inf), neg)
        return jnp.where(lane == i, a, sel)

    self0 = jnp.zeros((_RT, KNN), jnp.float32)
    sel = lax.fori_loop(0, KNN, it, self0)
    idx_ref[...] = sel.astype(jnp.int32) + b * NPB


def _knn(xi, yi, zi, xj, yj, zj, nb):
    bi = pl.BlockSpec((None, _RT, 1), lambda b, t: (b, t, 0))
    bj = pl.BlockSpec((None, 1, NPB), lambda b, t: (b, 0, 0))
    bo = pl.BlockSpec((None, _RT, KNN), lambda b, t: (b, t, 0))
    return pl.pallas_call(
        _knn_body,
        grid=(nb, NPB // _RT),
        in_specs=[bi, bi, bi, bj, bj, bj],
        out_specs=bo,
        out_shape=jax.ShapeDtypeStruct((nb, NPB, KNN), jnp.int32),
        scratch_shapes=[pltpu.VMEM((_RT, NPB), jnp.float32)],
    )(xi, yi, zi, xj, yj, zj)


_RM = 1024


def _uv_body(f_ref, w_ref, b_ref, u_ref, v_ref):
    f = f_ref[...]
    w = w_ref[...]
    wa = w[:, :C]
    wb = w[:, C:]
    dn = (((1,), (1,)), ((), ()))
    u_ref[...] = lax.dot_general(f, wa - wb, dn,
                                 preferred_element_type=jnp.float32) + b_ref[...]
    v_ref[...] = lax.dot_general(f, wb, dn, preferred_element_type=jnp.float32)


def _uv(f, w, bvec):
    nr = f.shape[0]
    return pl.pallas_call(
        _uv_body,
        grid=(nr // _RM,),
        in_specs=[pl.BlockSpec((_RM, C), lambda i: (i, 0)),
                  pl.BlockSpec((C, 2 * C), lambda i: (0, 0)),
                  pl.BlockSpec((1, C), lambda i: (0, 0))],
        out_specs=[pl.BlockSpec((_RM, C), lambda i: (i, 0)),
                   pl.BlockSpec((_RM, C), lambda i: (i, 0))],
        out_shape=[jax.ShapeDtypeStruct((nr, C), jnp.float32),
                   jax.ShapeDtypeStruct((nr, C), jnp.float32)],
    )(f, w, bvec.reshape(1, C))


_NC = 2
_NS = 16
_NW = _NC * _NS
_CHR = 32
_IW = 64
_NG = _CHR * KNN // _IW
_NBUF = 2


def _make_sc(nrows, add_base):
    _ROWS_W = nrows // _NW
    _NCHUNK = _ROWS_W // _CHR
    assert _NCHUNK % _NBUF == 0
    mesh = plsc.VectorSubcoreMesh(core_axis_name="c", subcore_axis_name="s",
                                  num_cores=_NC)
    scratch = [
        pltpu.VMEM((_NBUF, _NG, _IW), jnp.int32),
        pltpu.VMEM((_NBUF, _CHR * KNN, C), jnp.float32),
        pltpu.VMEM((_NBUF, _CHR, C), jnp.float32),
        pltpu.VMEM((_NBUF, _CHR, C), jnp.float32),
        pltpu.VMEM((_NBUF, _CHR, C), jnp.float32),
    ]
    if add_base:
        scratch.append(pltpu.VMEM((_NBUF, _CHR, C), jnp.float32))
    scratch += [pltpu.SemaphoreType.DMA] * (3 * _NBUF)

    def body(u_hbm, v_hbm, f_hbm, idx_hbm, *rest):
        if add_base:
            (base_hbm, out_hbm, idx_v, rows_v, u_v, f_v, o_v, base_v,
             *sems) = rest
        else:
            (out_hbm, idx_v, rows_v, u_v, f_v, o_v, *sems) = rest
        gsem = sems[:_NBUF]
        lsem = sems[_NBUF:2 * _NBUF]
        osem = sems[2 * _NBUF:]
        wid = lax.axis_index("s") * _NC + lax.axis_index("c")
        r0w = wid * _ROWS_W

        def _in_copies(cidx, b):
            r0 = pl.multiple_of(r0w + cidx * _CHR, _CHR)
            cps = [pltpu.make_async_copy(v_hbm.at[idx_v.at[b, g]],
                                         rows_v.at[b, pl.ds(g * _IW, _IW)],
                                         gsem[b])
                   for g in range(_NG)]
            cps.append(pltpu.make_async_copy(u_hbm.at[pl.ds(r0, _CHR)],
                                             u_v.at[b], lsem[b]))
            cps.append(pltpu.make_async_copy(f_hbm.at[pl.ds(r0, _CHR)],
                                             f_v.at[b], lsem[b]))
            if add_base:
                cps.append(pltpu.make_async_copy(
                    base_hbm.at[pl.ds(r0, _CHR)], base_v.at[b], lsem[b]))
            return cps

        def stage(cidx, b):
            r0 = pl.multiple_of(r0w + cidx * _CHR, _CHR)
            pltpu.sync_copy(
                idx_hbm.at[pl.ds(pl.multiple_of(r0 * KNN // _IW, 8), _NG)],
                idx_v.at[b])
            for cp in _in_copies(cidx, b):
                cp.start()

        def compute(cidx, b):
            for cp in _in_copies(cidx, b):
                cp.wait()

            def rowfn(r2, rc):
                for u in range(2):
                    r = r2 * 2 + u
                    rk = r * KNN
                    for j in range(C // 16):
                        sl = pl.ds(j * 16, 16)
                        m = rows_v[b, rk, sl]
                        for k in range(1, KNN):
                            m = jnp.maximum(m, rows_v[b, rk + k, sl])
                        agg = jnp.maximum(u_v[b, r, sl] + m, 0.0)
                        fn = jnp.maximum(f_v[b, r, sl] + agg, 0.0)
                        if add_base:
                            fn = fn + base_v[b, r, sl]
                        o_v[b, r, sl] = fn
                return rc

            lax.fori_loop(0, _CHR // 2, rowfn, 0)

        def _out_copy(cidx, b):
            r0 = pl.multiple_of(r0w + cidx * _CHR, _CHR)
            return pltpu.make_async_copy(o_v.at[b],
                                         out_hbm.at[pl.ds(r0, _CHR)], osem[b])

        for b in range(_NBUF):
            stage(b, b)

        def group(g0, carry):
            for b in range(_NBUF):
                cidx = g0 * _NBUF + b

                @pl.when(cidx >= _NBUF)
                def _():
                    _out_copy(cidx - _NBUF, b).wait()
                compute(cidx, b)
                _out_copy(cidx, b).start()

                @pl.when(cidx + _NBUF < _NCHUNK)
                def _():
                    stage(cidx + _NBUF, b)
            return carry

        lax.fori_loop(0, _NCHUNK // _NBUF, group, 0)
        for b in range(_NBUF):
            _out_copy(_NCHUNK - _NBUF + b, b).wait()

    return pl.kernel(
        body,
        mesh=mesh,
        out_type=jax.ShapeDtypeStruct((nrows, C), jnp.float32),
        scratch_types=scratch,
        compiler_params=pltpu.CompilerParams(use_tc_tiling_on_sc=False),
    )


_G = 1


def kernel(voxel_coords, pillar_features, W1, b1, W2, b2):
    nbg = NB // _G
    ng = N // _G
    _sc_l1 = _make_sc(ng, False)
    _sc_l2 = _make_sc(ng, True)
    pos = voxel_coords[:, 1:4]
    p3 = pos.reshape(NB, NPB, 3)
    xi = p3[:, :, 0:1]
    yi = p3[:, :, 1:2]
    zi = p3[:, :, 2:3]
    xj = xi.reshape(NB, 1, NPB)
    yj = yi.reshape(NB, 1, NPB)
    zj = zi.reshape(NB, 1, NPB)
    outs = []
    for g in range(_G):
        bs = slice(g * nbg, (g + 1) * nbg)
        idx = _knn(xi[bs], yi[bs], zi[bs], xj[bs], yj[bs], zj[bs], nbg)
        idx2d = idx.reshape(ng * KNN // _IW, _IW)
        f0 = pillar_features[g * ng:(g + 1) * ng]
        u1, v1 = _uv(f0, W1, b1)
        f1 = _sc_l1(u1, v1, f0, idx2d)
        u2, v2 = _uv(f1, W2, b2)
        outs.append(_sc_l2(u2, v2, f1, idx2d, f0))
    return jnp.concatenate(outs, axis=0)
